# baseline (speedup 1.0000x reference)
import jax
import jax.numpy as jnp
from jax import lax
from jax.experimental import pallas as pl
from jax.experimental.pallas import tpu as pltpu

N = 2048
B = 4
NP = 512
M = 9
C = 128
CM = 256
K_SEL = 256
NM = N * M
GK = NP * M

bf16 = jnp.bfloat16


def _leaky(x):
    return jnp.where(x >= 0, x, 0.01 * x)


def _bf(x):
    return x.astype(bf16).astype(jnp.float32)


def _compw_kernel(w_ref, wct_ref, bc_ref, o_ref):
    o_ref[...] = lax.dot_general(
        w_ref[...].astype(bf16), wct_ref[...].astype(bf16),
        (((1,), (0,)), ((), ())),
        preferred_element_type=jnp.float32) + bc_ref[0:1, :]


def _conv_kernel(pos_ref, cpt_ref, qn_ref, pn_ref, cw_ref, y_ref, kbf_ref):
    g = pl.program_id(0)
    kbf_ref[...] = jnp.zeros((NP, NM), jnp.float32)
    px = _bf(pos_ref[:, 0:1])
    py = _bf(pos_ref[:, 1:2])
    qn = qn_ref[...]
    base = g * GK
    for t in range(M):
        sl = pl.ds(base + t * NP, NP)
        cx = _bf(cpt_ref[0:1, sl])
        cy = _bf(cpt_ref[1:2, sl])
        cross = px * cx + py * cy
        d2 = (qn + pn_ref[0:1, sl]) - 2.0 * cross
        kbf_ref[:, pl.ds(base + t * NP, NP)] = _bf(jnp.exp(d2 * -2.0))
    acc = lax.dot_general(kbf_ref[...], cw_ref[...].astype(jnp.float32),
                          (((1,), (0,)), ((), ())),
                          preferred_element_type=jnp.float32,
                          precision=lax.Precision.HIGHEST)
    y_ref[...] = _leaky(acc)


def _bn(x, gamma, beta):
    mu = jnp.mean(x, axis=0, keepdims=True)
    var = jnp.mean((x - mu) ** 2, axis=0, keepdims=True)
    return (x - mu) / jnp.sqrt(var + 1e-5) * gamma + beta


def _bitonic_desc(keys, idxs):
    lane = lax.broadcasted_iota(jnp.int32, (B, NP), 1)
    k = 2
    while k <= NP:
        j = k // 2
        while j >= 1:
            bit_clear = (lane & j) == 0
            pk = jnp.where(bit_clear,
                           pltpu.roll(keys, NP - j, 1), pltpu.roll(keys, j, 1))
            pi = jnp.where(bit_clear,
                           pltpu.roll(idxs, NP - j, 1), pltpu.roll(idxs, j, 1))
            take_max = bit_clear ^ ((lane & k) != 0)
            self_wins = (keys > pk) | ((keys == pk) & (idxs < pi))
            use_self = self_wins == take_max
            keys = jnp.where(use_self, keys, pk)
            idxs = jnp.where(use_self, idxs, pi)
            j //= 2
        k *= 2
    return keys, idxs


def _tail_kernel(y_ref, w_in_ref, pos_ref, w1_ref, b1_ref, g1_ref, be1_ref,
                 g2_ref, be2_ref, w2_ref, b2_ref, psel_ref, pnorm_ref,
                 pos_out_ref, w_sel_ref):
    y = _bn(y_ref[...], g1_ref[0:1, :], be1_ref[0:1, :]) + w_in_ref[...]
    h = _leaky(lax.dot_general(y.astype(bf16), w1_ref[...].astype(bf16),
                               (((1,), (0,)), ((), ())),
                               preferred_element_type=jnp.float32)
               + b1_ref[0:1, :])
    h = _bn(h, g2_ref[0:1, :], be2_ref[0:1, :])
    w_out = y + (lax.dot_general(h.astype(bf16), w2_ref[...].astype(bf16),
                                 (((1,), (0,)), ((), ())),
                                 preferred_element_type=jnp.float32)
                 + b2_ref[0:1, :])

    psel_bf = psel_ref[0:1, :].astype(bf16)
    pnorm = pnorm_ref[0, 0]
    rows = []
    for g in range(B):
        wg_t = jnp.transpose(w_out[g * NP:(g + 1) * NP, :])
        rows.append(lax.dot_general(psel_bf, wg_t.astype(bf16),
                                    (((1,), (0,)), ((), ())),
                                    preferred_element_type=jnp.float32))
    score = jnp.tanh(jnp.concatenate(rows, axis=0) / pnorm)

    keys, idxs = _bitonic_desc(score,
                               lax.broadcasted_iota(jnp.int32, (B, NP), 1))
    sv = keys[:, :K_SEL]
    si = idxs[:, :K_SEL]

    col = lax.broadcasted_iota(jnp.int32, (NP, K_SEL), 0)
    for g in range(B):
        oh = (col == si[g:g + 1, :]).astype(jnp.float32)
        wg = w_out[g * NP:(g + 1) * NP, :]
        pg = pos_ref[g * NP:(g + 1) * NP, :]
        gathered = lax.dot_general(oh, wg, (((0,), (0,)), ((), ())),
                                   preferred_element_type=jnp.float32,
                                   precision=lax.Precision.HIGHEST)
        sv_col = jnp.transpose(sv[g:g + 1, :])
        w_sel_ref[g * K_SEL:(g + 1) * K_SEL, :] = gathered * sv_col
        pos_out_ref[g * K_SEL:(g + 1) * K_SEL, :] = lax.dot_general(
            oh, pg, (((0,), (0,)), ((), ())),
            preferred_element_type=jnp.float32,
            precision=lax.Precision.HIGHEST)


@jax.jit
def kernel(positions, weights, batch, filter_pos, Wc, bc, gamma1, beta1,
           W1, b1, gamma2, beta2, W2, b2, p_sel):
    comp_pos = (positions[:, None, :] + filter_pos[None, :, :]).reshape(NM, 2)
    cpt = comp_pos.T
    qn = jnp.sum(positions ** 2, axis=-1)[:, None]
    pn = jnp.sum(comp_pos ** 2, axis=-1)[None, :]
    wct = Wc.transpose(1, 0, 2).reshape(C, M * C)
    pnorm = jnp.linalg.norm(p_sel).reshape(1, 1)

    comp_w = pl.pallas_call(
        _compw_kernel,
        in_specs=[
            pl.BlockSpec((N, C), lambda: (0, 0)),
            pl.BlockSpec((C, M * C), lambda: (0, 0)),
            pl.BlockSpec((1, M * C), lambda: (0, 0)),
        ],
        out_specs=pl.BlockSpec((N, M * C), lambda: (0, 0)),
        out_shape=jax.ShapeDtypeStruct((N, M * C), jnp.float32),
    )(weights, wct, jnp.tile(bc, M).reshape(1, M * C))
    cw_bf = comp_w.reshape(NM, C).astype(bf16)

    y = pl.pallas_call(
        _conv_kernel,
        grid=(B,),
        in_specs=[
            pl.BlockSpec((NP, 2), lambda g: (g, 0)),
            pl.BlockSpec((2, NM), lambda g: (0, 0)),
            pl.BlockSpec((NP, 1), lambda g: (g, 0)),
            pl.BlockSpec((1, NM), lambda g: (0, 0)),
            pl.BlockSpec((NM, C), lambda g: (0, 0)),
        ],
        out_specs=pl.BlockSpec((NP, C), lambda g: (g, 0)),
        out_shape=jax.ShapeDtypeStruct((N, C), jnp.float32),
        scratch_shapes=[pltpu.VMEM((NP, NM), jnp.float32)],
        compiler_params=pltpu.CompilerParams(
            vmem_limit_bytes=100 * 1024 * 1024),
    )(positions, cpt, qn, pn, cw_bf)

    pos_out, w_sel = pl.pallas_call(
        _tail_kernel,
        in_specs=[
            pl.BlockSpec((N, C), lambda: (0, 0)),
            pl.BlockSpec((N, C), lambda: (0, 0)),
            pl.BlockSpec((N, 2), lambda: (0, 0)),
            pl.BlockSpec((C, CM), lambda: (0, 0)),
            pl.BlockSpec((1, CM), lambda: (0, 0)),
            pl.BlockSpec((1, C), lambda: (0, 0)),
            pl.BlockSpec((1, C), lambda: (0, 0)),
            pl.BlockSpec((1, CM), lambda: (0, 0)),
            pl.BlockSpec((1, CM), lambda: (0, 0)),
            pl.BlockSpec((CM, C), lambda: (0, 0)),
            pl.BlockSpec((1, C), lambda: (0, 0)),
            pl.BlockSpec((1, C), lambda: (0, 0)),
            pl.BlockSpec(memory_space=pltpu.SMEM),
        ],
        out_specs=[
            pl.BlockSpec((B * K_SEL, 2), lambda: (0, 0)),
            pl.BlockSpec((B * K_SEL, C), lambda: (0, 0)),
        ],
        out_shape=[
            jax.ShapeDtypeStruct((B * K_SEL, 2), jnp.float32),
            jax.ShapeDtypeStruct((B * K_SEL, C), jnp.float32),
        ],
    )(y, weights, positions, W1, b1.reshape(1, CM), gamma1.reshape(1, C),
      beta1.reshape(1, C), gamma2.reshape(1, CM), beta2.reshape(1, CM),
      W2, b2.reshape(1, C), p_sel.reshape(1, C), pnorm)

    batch_out = jnp.full((B,), K_SEL, dtype=batch.dtype)
    return pos_out, w_sel, batch_out
